# baseline (speedup 1.0000x reference)
import functools
import math

import jax
import jax.numpy as jnp
from jax import lax
from jax.experimental import pallas as pl
from jax.experimental.pallas import tpu as pltpu
from jax.experimental.pallas import tpu_sc as plsc

EMBED_DIM = 64
_SCALE = math.sqrt(EMBED_DIM)

_INFO = plsc.get_sparse_core_info()
_NC = _INFO.num_cores
_NS = _INFO.num_subcores
_NW = _NC * _NS
_GRP = 128
_NB = 8
_LOOK = 4


def _make_kernel(n_tokens: int):
    assert n_tokens % (_NW * _GRP) == 0
    per_w = n_tokens // _NW
    n_grp = per_w // _GRP
    assert n_grp % _NB == 0 and n_grp > _NB

    mesh = plsc.VectorSubcoreMesh(core_axis_name="c", subcore_axis_name="s")

    @functools.partial(
        pl.kernel,
        mesh=mesh,
        compiler_params=pltpu.CompilerParams(use_tc_tiling_on_sc=False),
        out_type=jax.ShapeDtypeStruct((n_tokens, EMBED_DIM), jnp.float32),
        scratch_types=(
            [pltpu.VMEM((n_grp, _GRP), jnp.int32)]
            + [pltpu.VMEM((_GRP, EMBED_DIM), jnp.float32)] * _NB
            + [pltpu.SemaphoreType.DMA] * (2 * _NB)
        ),
    )
    def k(tok_hbm, tab_hbm, out_hbm, idx_v, *scratch):
        rbufs = scratch[:_NB]
        gsems = scratch[_NB:2 * _NB]
        ssems = scratch[2 * _NB:]
        wid = lax.axis_index("s") * _NC + lax.axis_index("c")
        pltpu.sync_copy(tok_hbm.at[wid], idx_v)

        def fire_gather(j, b):
            pltpu.async_copy(tab_hbm.at[idx_v.at[j]], rbufs[b], gsems[b])

        def wait_gather(b):
            pltpu.make_async_copy(
                tab_hbm.at[pl.ds(0, _GRP)], rbufs[b], gsems[b]).wait()

        def fire_scatter(j, b):
            pltpu.async_copy(
                rbufs[b], out_hbm.at[pl.ds(wid * per_w + j * _GRP, _GRP)],
                ssems[b])

        def wait_scatter(b):
            pltpu.make_async_copy(
                rbufs[b], out_hbm.at[pl.ds(0, _GRP)], ssems[b]).wait()

        def scale(buf):
            def row(r, _):
                for v in range(EMBED_DIM // 16):
                    sl = pl.ds(v * 16, 16)
                    buf[r, sl] = buf[r, sl] * _SCALE
                return 0
            lax.fori_loop(0, _GRP, row, 0)

        for j in range(_LOOK):
            fire_gather(j, j % _NB)

        def outer(t, _):
            for b in range(_NB):
                j = t * _NB + b
                wait_gather(b)
                scale(rbufs[b])
                fire_scatter(j, b)
                b2 = (b + _LOOK) % _NB

                @pl.when(j >= _NB - _LOOK)
                def _():
                    wait_scatter(b2)

                @pl.when(j + _LOOK < n_grp)
                def _():
                    fire_gather(j + _LOOK, b2)
            return 0

        lax.fori_loop(0, n_grp // _NB, outer, 0)

        for j in range(n_grp - _NB + _LOOK, n_grp):
            wait_scatter(j % _NB)

    return k


def kernel(tokens, embedding_weight):
    b, t = tokens.shape
    n_tokens = b * t
    idx = tokens.reshape(_NW, n_tokens // (_NW * _GRP), _GRP).astype(jnp.int32)
    out = _make_kernel(n_tokens)(idx, embedding_weight)
    return out.reshape(b, t, EMBED_DIM)


# speedup vs baseline: 1.2773x; 1.2773x over previous
import functools
import math

import jax
import jax.numpy as jnp
from jax import lax
from jax.experimental import pallas as pl
from jax.experimental.pallas import tpu as pltpu
from jax.experimental.pallas import tpu_sc as plsc

EMBED_DIM = 64
_SCALE = math.sqrt(EMBED_DIM)

_INFO = plsc.get_sparse_core_info()
_NC = _INFO.num_cores
_NS = _INFO.num_subcores
_NW = _NC * _NS
_GRP = 128
_NB = 4
_LOOK = 2


def _make_kernel(n_tokens: int):
    assert n_tokens % (_NW * _GRP) == 0
    per_w = n_tokens // _NW
    n_grp = per_w // _GRP
    assert n_grp % _NB == 0 and n_grp > _NB

    mesh = plsc.VectorSubcoreMesh(core_axis_name="c", subcore_axis_name="s")

    @functools.partial(
        pl.kernel,
        mesh=mesh,
        compiler_params=pltpu.CompilerParams(use_tc_tiling_on_sc=False),
        out_type=jax.ShapeDtypeStruct((n_tokens, 2 * EMBED_DIM), jnp.float32),
        scratch_types=(
            [pltpu.VMEM((per_w,), jnp.int32)]
            + [pltpu.VMEM((_GRP, EMBED_DIM), jnp.float32)] * _NB
            + [pltpu.SemaphoreType.DMA] * (2 * _NB)
        ),
    )
    def k(tok_hbm, tab_hbm, out_hbm, idx_v, *scratch):
        rbufs = scratch[:_NB]
        gsems = scratch[_NB:2 * _NB]
        ssems = scratch[2 * _NB:]
        wid = lax.axis_index("s") * _NC + lax.axis_index("c")
        pltpu.sync_copy(tok_hbm.at[wid], idx_v)

        def fire_gather(j, b):
            pltpu.async_copy(
                tab_hbm.at[idx_v.at[pl.ds(j * _GRP, _GRP)]], rbufs[b],
                gsems[b])

        def wait_gather(b):
            pltpu.make_async_copy(
                tab_hbm.at[pl.ds(0, _GRP)], rbufs[b], gsems[b]).wait()

        def fire_scatter(j, b):
            pltpu.async_copy(
                rbufs[b],
                out_hbm.at[pl.ds(wid * per_w + j * _GRP, _GRP),
                           pl.ds(0, EMBED_DIM)],
                ssems[b])

        def wait_scatter(b):
            pltpu.make_async_copy(
                rbufs[b],
                out_hbm.at[pl.ds(0, _GRP), pl.ds(0, EMBED_DIM)],
                ssems[b]).wait()

        def scale(buf):
            def row(r, _):
                for v in range(EMBED_DIM // 16):
                    sl = pl.ds(v * 16, 16)
                    buf[r, sl] = buf[r, sl] * _SCALE
                return 0
            lax.fori_loop(0, _GRP, row, 0)

        for j in range(_LOOK):
            fire_gather(j, j % _NB)

        def outer(t, _):
            for b in range(_NB):
                j = t * _NB + b
                wait_gather(b)

                @pl.when(j >= _NB)
                def _():
                    wait_scatter(b)

                scale(rbufs[b])
                fire_scatter(j, b)
                b2 = (b + _LOOK) % _NB

                @pl.when(j + _LOOK < n_grp)
                def _():
                    fire_gather(j + _LOOK, b2)
            return 0

        lax.fori_loop(0, n_grp // _NB, outer, 0)

        for j in range(n_grp - _NB, n_grp):
            wait_scatter(j % _NB)

    return k


def kernel(tokens, embedding_weight):
    b, t = tokens.shape
    n_tokens = b * t
    idx = tokens.reshape(_NW, n_tokens // _NW).astype(jnp.int32)
    out = _make_kernel(n_tokens)(idx, embedding_weight)
    return out[:, :EMBED_DIM].reshape(b, t, EMBED_DIM)


# speedup vs baseline: 1.3336x; 1.0440x over previous
import functools
import math

import jax
import jax.numpy as jnp
from jax import lax
from jax.experimental import pallas as pl
from jax.experimental.pallas import tpu as pltpu
from jax.experimental.pallas import tpu_sc as plsc

EMBED_DIM = 64
_SCALE = math.sqrt(EMBED_DIM)

_INFO = plsc.get_sparse_core_info()
_NC = _INFO.num_cores
_NS = _INFO.num_subcores
_NW = _NC * _NS
_GRP = 128
_NB = 8
_LOOK = 4


def _make_kernel(n_tokens: int):
    assert n_tokens % (_NW * _GRP) == 0
    per_w = n_tokens // _NW
    n_grp = per_w // _GRP
    assert n_grp % _NB == 0 and n_grp > _NB

    mesh = plsc.VectorSubcoreMesh(core_axis_name="c", subcore_axis_name="s")

    @functools.partial(
        pl.kernel,
        mesh=mesh,
        compiler_params=pltpu.CompilerParams(use_tc_tiling_on_sc=False),
        out_type=jax.ShapeDtypeStruct((n_tokens, 2 * EMBED_DIM), jnp.float32),
        scratch_types=(
            [pltpu.VMEM((per_w,), jnp.int32)]
            + [pltpu.VMEM((_GRP, EMBED_DIM), jnp.float32)] * _NB
            + [pltpu.SemaphoreType.DMA] * (2 * _NB)
        ),
    )
    def k(tok_hbm, tab_hbm, out_hbm, idx_v, *scratch):
        rbufs = scratch[:_NB]
        gsems = scratch[_NB:2 * _NB]
        ssems = scratch[2 * _NB:]
        wid = lax.axis_index("s") * _NC + lax.axis_index("c")
        pltpu.sync_copy(tok_hbm.at[wid], idx_v)

        def fire_gather(j, b):
            pltpu.async_copy(
                tab_hbm.at[idx_v.at[pl.ds(j * _GRP, _GRP)]], rbufs[b],
                gsems[b])

        def wait_gather(b):
            pltpu.make_async_copy(
                tab_hbm.at[pl.ds(0, _GRP)], rbufs[b], gsems[b]).wait()

        def fire_scatter(j, b):
            pltpu.async_copy(
                rbufs[b],
                out_hbm.at[pl.ds(wid * per_w + j * _GRP, _GRP),
                           pl.ds(0, EMBED_DIM)],
                ssems[b])

        def wait_scatter(b):
            pltpu.make_async_copy(
                rbufs[b],
                out_hbm.at[pl.ds(0, _GRP), pl.ds(0, EMBED_DIM)],
                ssems[b]).wait()

        def scale(buf):
            def row(r, _):
                for u in range(2):
                    for v in range(EMBED_DIM // 16):
                        sl = pl.ds(v * 16, 16)
                        buf[r * 2 + u, sl] = buf[r * 2 + u, sl] * _SCALE
                return 0
            lax.fori_loop(0, _GRP // 2, row, 0)

        for j in range(_LOOK):
            fire_gather(j, j % _NB)

        def outer(t, _):
            for b in range(_NB):
                j = t * _NB + b
                wait_gather(b)

                @pl.when(j >= _NB)
                def _():
                    wait_scatter(b)

                scale(rbufs[b])
                fire_scatter(j, b)
                b2 = (b + _LOOK) % _NB

                @pl.when(j + _LOOK < n_grp)
                def _():
                    fire_gather(j + _LOOK, b2)
            return 0

        lax.fori_loop(0, n_grp // _NB, outer, 0)

        for j in range(n_grp - _NB, n_grp):
            wait_scatter(j % _NB)

    return k


def kernel(tokens, embedding_weight):
    b, t = tokens.shape
    n_tokens = b * t
    idx = tokens.reshape(_NW, n_tokens // _NW).astype(jnp.int32)
    out = _make_kernel(n_tokens)(idx, embedding_weight)
    return out[:, :EMBED_DIM].reshape(b, t, EMBED_DIM)


# speedup vs baseline: 1.3338x; 1.0002x over previous
import functools
import math

import jax
import jax.numpy as jnp
from jax import lax
from jax.experimental import pallas as pl
from jax.experimental.pallas import tpu as pltpu
from jax.experimental.pallas import tpu_sc as plsc

EMBED_DIM = 64
_SCALE = math.sqrt(EMBED_DIM)

_INFO = plsc.get_sparse_core_info()
_NC = _INFO.num_cores
_NS = _INFO.num_subcores
_NW = _NC * _NS
_GRP = 128
_NB = 8
_LOOK = 4


def _make_kernel(n_tokens: int):
    assert n_tokens % (_NW * _GRP) == 0
    per_w = n_tokens // _NW
    n_grp = per_w // _GRP
    assert n_grp % _NB == 0 and n_grp > _NB

    mesh = plsc.VectorSubcoreMesh(core_axis_name="c", subcore_axis_name="s")

    @functools.partial(
        pl.kernel,
        mesh=mesh,
        compiler_params=pltpu.CompilerParams(use_tc_tiling_on_sc=False),
        out_type=jax.ShapeDtypeStruct((n_tokens, 2 * EMBED_DIM), jnp.float32),
        scratch_types=(
            [pltpu.VMEM((per_w,), jnp.int32)]
            + [pltpu.VMEM((_GRP, EMBED_DIM), jnp.float32)] * _NB
            + [pltpu.SemaphoreType.DMA] * (2 * _NB)
        ),
    )
    def k(tok_hbm, tab_hbm, out_hbm, idx_v, *scratch):
        rbufs = scratch[:_NB]
        gsems = scratch[_NB:2 * _NB]
        ssems = scratch[2 * _NB:]
        wid = lax.axis_index("s") * _NC + lax.axis_index("c")
        pltpu.sync_copy(tok_hbm.at[wid], idx_v)

        def fire_gather(j, b):
            pltpu.async_copy(
                tab_hbm.at[idx_v.at[pl.ds(j * _GRP, _GRP)]], rbufs[b],
                gsems[b])

        def wait_gather(b):
            pltpu.make_async_copy(
                tab_hbm.at[pl.ds(0, _GRP)], rbufs[b], gsems[b]).wait()

        def fire_scatter(j, b):
            pltpu.async_copy(
                rbufs[b],
                out_hbm.at[pl.ds(wid * per_w + j * _GRP, _GRP),
                           pl.ds(0, EMBED_DIM)],
                ssems[b])

        def wait_scatter(b):
            pltpu.make_async_copy(
                rbufs[b],
                out_hbm.at[pl.ds(0, _GRP), pl.ds(0, EMBED_DIM)],
                ssems[b]).wait()

        def scale(buf):
            def row(r, _):
                for u in range(2):
                    for v in range(EMBED_DIM // 16):
                        sl = pl.ds(v * 16, 16)
                        buf[r * 2 + u, sl] = buf[r * 2 + u, sl] * _SCALE
                return 0
            lax.fori_loop(0, _GRP // 2, row, 0)

        for j in range(_LOOK):
            fire_gather(j, j % _NB)

        def outer(t, _):
            for b in range(_NB):
                j = t * _NB + b
                wait_gather(b)
                scale(rbufs[b])
                fire_scatter(j, b)
                b2 = (b + _LOOK) % _NB

                @pl.when(j >= _NB - _LOOK)
                def _():
                    wait_scatter(b2)

                @pl.when(j + _LOOK < n_grp)
                def _():
                    fire_gather(j + _LOOK, b2)
            return 0

        lax.fori_loop(0, n_grp // _NB, outer, 0)

        for j in range(n_grp - _NB + _LOOK, n_grp):
            wait_scatter(j % _NB)

    return k


def kernel(tokens, embedding_weight):
    b, t = tokens.shape
    n_tokens = b * t
    idx = tokens.reshape(_NW, n_tokens // _NW).astype(jnp.int32)
    out = _make_kernel(n_tokens)(idx, embedding_weight)
    return out[:, :EMBED_DIM].reshape(b, t, EMBED_DIM)
